# baseline (speedup 1.0000x reference)
import jax
import jax.numpy as jnp
from jax.experimental import pallas as pl
from jax.experimental.pallas import tpu as pltpu

def _pos_body(msk_ref, pos_ref):
    e = msk_ref[...].reshape(8, 128, msk_ref.shape[1]).astype(jnp.int32)
    giota = jax.lax.broadcasted_iota(jnp.int32, e.shape, 2) * 4
    contrib = jnp.where(e > 0, giota + e, 0)
    pos_ref[...] = (jnp.sum(contrib, axis=2) - 1)[None]


def _remap_body(pos_ref, lay_ref, out_ref):
    j = pl.program_id(1)
    p = pos_ref[...]
    x = p & 3
    cell = p >> 2
    clear_sw = (x & 1) == 0
    clear_sh = x < 2
    v = lay_ref[...].reshape(128, 32, 128)
    ciota = jax.lax.broadcasted_iota(jnp.int32, (128, 32, 128), 0) + j * 128
    is_sw = v == 11
    is_sh = v == 12
    ch1 = jnp.where(v == 9, 4, 0)
    ch1 = jnp.where(is_sw, 3, ch1)
    ch1 = jnp.where(is_sh, 2, ch1)
    ch1 = jnp.where(v == 13, 1, ch1)
    ch0 = jnp.where(is_sw & clear_sw, 1, v)
    ch0 = jnp.where(is_sh & clear_sh, 1, ch0)
    ch0 = jnp.where(ciota == cell, 10, ch0)

    t = jnp.stack([ch0, ch1], axis=1)
    t = t.reshape(4, 32, 2, 32, 128)
    t = t.transpose(0, 2, 1, 3, 4)
    out_ref[...] = t.reshape(8, 32, 32, 128).astype(jnp.uint8)


def kernel(layout, mask_agent_ijx):
    b, h, w, _ = layout.shape
    hw = h * w
    q = b // 1024
    q4 = b // 4096
    lay = layout.transpose(1, 2, 3, 0).reshape(hw, q4, 32, 128)
    w4 = jnp.asarray([1, 2, 3, 4], dtype=jnp.int8)
    msk = jnp.sum(
        mask_agent_ijx.reshape(b, hw, 4).astype(jnp.int8) * w4,
        axis=-1,
        dtype=jnp.int8,
    )

    pos = pl.pallas_call(
        _pos_body,
        grid=(q,),
        in_specs=[pl.BlockSpec((1024, hw), lambda i: (i, 0))],
        out_specs=pl.BlockSpec((1, 8, 128), lambda i: (i, 0, 0)),
        out_shape=jax.ShapeDtypeStruct((q, 8, 128), jnp.int32),
        compiler_params=pltpu.CompilerParams(allow_input_fusion=[True]),
    )(msk)
    pos4 = pos.reshape(q4, 32, 128)

    out8 = pl.pallas_call(
        _remap_body,
        grid=(q4, hw // 128),
        in_specs=[
            pl.BlockSpec((1, 32, 128), lambda i, j: (i, 0, 0)),
            pl.BlockSpec((128, 1, 32, 128), lambda i, j: (j, i, 0, 0)),
        ],
        out_specs=pl.BlockSpec((8, 32, 32, 128), lambda i, j: (j, 0, i, 0)),
        out_shape=jax.ShapeDtypeStruct((2 * h, w, b // 128, 128), jnp.uint8),
    )(pos4, lay)

    u = out8.reshape(h, 2, w, b // 128, 128)
    u = u.transpose(3, 4, 0, 2, 1)
    return u.reshape(b, h, w, 2)


# speedup vs baseline: 2.7584x; 2.7584x over previous
import jax
import jax.numpy as jnp
from jax.experimental import pallas as pl
from jax.experimental.pallas import tpu as pltpu

def _pos_body(msk_ref, pos_ref):
    m = msk_ref[...].reshape(8, 128, msk_ref.shape[1])
    iota = jax.lax.broadcasted_iota(jnp.int32, m.shape, 2)
    pos_ref[...] = jnp.sum(m.astype(jnp.int32) * iota, axis=2)[None]


def _remap_body(pos_ref, lay_ref, out_ref):
    j = pl.program_id(1)
    p = pos_ref[...]
    x = p & 3
    cell = p >> 2
    clear_sw = (x & 1) == 0
    clear_sh = x < 2
    v = lay_ref[...].reshape(128, 32, 128)
    ciota = jax.lax.broadcasted_iota(jnp.int32, (128, 32, 128), 0) + j * 128
    is_sw = v == 11
    is_sh = v == 12
    ch1 = jnp.where(v == 9, 4, 0)
    ch1 = jnp.where(is_sw, 3, ch1)
    ch1 = jnp.where(is_sh, 2, ch1)
    ch1 = jnp.where(v == 13, 1, ch1)
    ch0 = jnp.where(is_sw & clear_sw, 1, v)
    ch0 = jnp.where(is_sh & clear_sh, 1, ch0)
    ch0 = jnp.where(ciota == cell, 10, ch0)

    t = jnp.stack([ch0, ch1], axis=1)
    t = t.reshape(4, 32, 2, 32, 128)
    t = t.transpose(0, 2, 1, 3, 4)
    out_ref[...] = t.reshape(8, 32, 32, 128).astype(jnp.uint8)


def kernel(layout, mask_agent_ijx):
    b, h, w, _ = layout.shape
    hw = h * w
    q = b // 1024
    q4 = b // 4096
    lay = layout.transpose(1, 2, 3, 0).reshape(hw, q4, 32, 128)
    msk = mask_agent_ijx.view(jnp.int8)

    pos = pl.pallas_call(
        _pos_body,
        grid=(q,),
        in_specs=[pl.BlockSpec((1024, 4 * hw), lambda i: (i, 0))],
        out_specs=pl.BlockSpec((1, 8, 128), lambda i: (i, 0, 0)),
        out_shape=jax.ShapeDtypeStruct((q, 8, 128), jnp.int32),
        compiler_params=pltpu.CompilerParams(allow_input_fusion=[True]),
    )(msk)
    pos4 = pos.reshape(q4, 32, 128)

    out8 = pl.pallas_call(
        _remap_body,
        grid=(q4, hw // 128),
        in_specs=[
            pl.BlockSpec((1, 32, 128), lambda i, j: (i, 0, 0)),
            pl.BlockSpec((128, 1, 32, 128), lambda i, j: (j, i, 0, 0)),
        ],
        out_specs=pl.BlockSpec((8, 32, 32, 128), lambda i, j: (j, 0, i, 0)),
        out_shape=jax.ShapeDtypeStruct((2 * h, w, b // 128, 128), jnp.uint8),
    )(pos4, lay)

    u = out8.reshape(h, 2, w, b // 128, 128)
    u = u.transpose(3, 4, 0, 2, 1)
    return u.reshape(b, h, w, 2)


# speedup vs baseline: 2.8452x; 1.0315x over previous
import jax
import jax.numpy as jnp
from jax.experimental import pallas as pl
from jax.experimental.pallas import tpu as pltpu

def _pos_body(msk_ref, pos_ref):
    m = msk_ref[...].reshape(8, 128, msk_ref.shape[1])
    iota = jax.lax.broadcasted_iota(jnp.int32, m.shape, 2)
    pos_ref[...] = jnp.sum(m.astype(jnp.int32) * iota, axis=2)[None]


def _remap_body(pos_ref, lay_ref, out_ref):
    j = pl.program_id(1)
    p = pos_ref[...]
    x = p & 3
    cell = p >> 2
    clear_sw = (x & 1) == 0
    clear_sh = x < 2
    v = lay_ref[...].reshape(256, 32, 128)
    ciota = jax.lax.broadcasted_iota(jnp.int32, (256, 32, 128), 0) + j * 256
    is_sw = v == 11
    is_sh = v == 12
    ch1 = jnp.where(v == 9, 4, 0)
    ch1 = jnp.where(is_sw, 3, ch1)
    ch1 = jnp.where(is_sh, 2, ch1)
    ch1 = jnp.where(v == 13, 1, ch1)
    ch0 = jnp.where(is_sw & clear_sw, 1, v)
    ch0 = jnp.where(is_sh & clear_sh, 1, ch0)
    ch0 = jnp.where(ciota == cell, 10, ch0)

    t = jnp.stack([ch0, ch1], axis=1)
    t = t.reshape(8, 32, 2, 32, 128)
    t = t.transpose(0, 2, 1, 3, 4)
    out_ref[...] = t.reshape(16, 32, 32, 128).astype(jnp.uint8)


def kernel(layout, mask_agent_ijx):
    b, h, w, _ = layout.shape
    hw = h * w
    q = b // 1024
    q4 = b // 4096
    lay = layout.transpose(1, 2, 3, 0).reshape(hw, q4, 32, 128)
    msk = mask_agent_ijx.view(jnp.int8)

    pos = pl.pallas_call(
        _pos_body,
        grid=(q,),
        in_specs=[pl.BlockSpec((1024, 4 * hw), lambda i: (i, 0))],
        out_specs=pl.BlockSpec((1, 8, 128), lambda i: (i, 0, 0)),
        out_shape=jax.ShapeDtypeStruct((q, 8, 128), jnp.int32),
        compiler_params=pltpu.CompilerParams(allow_input_fusion=[True]),
    )(msk)
    pos4 = pos.reshape(q4, 32, 128)

    out8 = pl.pallas_call(
        _remap_body,
        grid=(q4, hw // 256),
        in_specs=[
            pl.BlockSpec((1, 32, 128), lambda i, j: (i, 0, 0)),
            pl.BlockSpec((256, 1, 32, 128), lambda i, j: (j, i, 0, 0)),
        ],
        out_specs=pl.BlockSpec((16, 32, 32, 128), lambda i, j: (j, 0, i, 0)),
        out_shape=jax.ShapeDtypeStruct((2 * h, w, b // 128, 128), jnp.uint8),
    )(pos4, lay)

    u = out8.reshape(h, 2, w, b // 128, 128)
    u = u.transpose(3, 4, 0, 2, 1)
    return u.reshape(b, h, w, 2)


# speedup vs baseline: 2.8727x; 1.0097x over previous
import jax
import jax.numpy as jnp
from jax.experimental import pallas as pl
from jax.experimental.pallas import tpu as pltpu

def _pos_body(msk_ref, pos_ref):
    m = msk_ref[...].reshape(8, 128, msk_ref.shape[1])
    iota = jax.lax.broadcasted_iota(jnp.int32, m.shape, 2)
    pos_ref[...] = jnp.sum(m.astype(jnp.int32) * iota, axis=2)[None]


def _remap_body(pos_ref, lay_ref, out_ref):
    j = pl.program_id(1)
    p = pos_ref[...]
    x = p & 3
    cell = p >> 2
    clear_sw = (x & 1) == 0
    clear_sh = x < 2
    v = lay_ref[...].reshape(512, 32, 128)
    ciota = jax.lax.broadcasted_iota(jnp.int32, (512, 32, 128), 0) + j * 512
    is_sw = v == 11
    is_sh = v == 12
    ch1 = jnp.where(v == 9, 4, 0)
    ch1 = jnp.where(is_sw, 3, ch1)
    ch1 = jnp.where(is_sh, 2, ch1)
    ch1 = jnp.where(v == 13, 1, ch1)
    ch0 = jnp.where(is_sw & clear_sw, 1, v)
    ch0 = jnp.where(is_sh & clear_sh, 1, ch0)
    ch0 = jnp.where(ciota == cell, 10, ch0)

    t = jnp.stack([ch0, ch1], axis=1)
    t = t.reshape(16, 32, 2, 32, 128)
    t = t.transpose(0, 2, 1, 3, 4)
    out_ref[...] = t.reshape(32, 32, 32, 128).astype(jnp.uint8)


def kernel(layout, mask_agent_ijx):
    b, h, w, _ = layout.shape
    hw = h * w
    q = b // 1024
    q4 = b // 4096
    lay = layout.transpose(1, 2, 3, 0).reshape(hw, q4, 32, 128)
    msk = mask_agent_ijx.view(jnp.int8)

    pos = pl.pallas_call(
        _pos_body,
        grid=(q,),
        in_specs=[pl.BlockSpec((1024, 4 * hw), lambda i: (i, 0))],
        out_specs=pl.BlockSpec((1, 8, 128), lambda i: (i, 0, 0)),
        out_shape=jax.ShapeDtypeStruct((q, 8, 128), jnp.int32),
        compiler_params=pltpu.CompilerParams(allow_input_fusion=[True]),
    )(msk)
    pos4 = pos.reshape(q4, 32, 128)

    out8 = pl.pallas_call(
        _remap_body,
        grid=(q4, hw // 512),
        in_specs=[
            pl.BlockSpec((1, 32, 128), lambda i, j: (i, 0, 0)),
            pl.BlockSpec((512, 1, 32, 128), lambda i, j: (j, i, 0, 0)),
        ],
        out_specs=pl.BlockSpec((32, 32, 32, 128), lambda i, j: (j, 0, i, 0)),
        out_shape=jax.ShapeDtypeStruct((2 * h, w, b // 128, 128), jnp.uint8),
    )(pos4, lay)

    u = out8.reshape(h, 2, w, b // 128, 128)
    u = u.transpose(3, 4, 0, 2, 1)
    return u.reshape(b, h, w, 2)
